# baseline (device time: 89574 ns/iter reference)
import jax
import jax.numpy as jnp
from jax import lax
from jax.experimental import pallas as pl
from jax.experimental.pallas import tpu as pltpu

M, N = 2048, 1024
HALF = M // 2
S4 = 4
C1 = HALF // S4
C2 = C1 // S4
C3 = C2 // 2
N_PHASES = 6


def kernel(x):
    x2 = x.reshape(M, N)

    def body(x_ref, out_ref,
             send_a, recv_a, send_b, recv_b, xrecv_a, xrecv_b,
             ssem_a, rsem_a, ssem_b, rsem_b,
             cred_a, cred_b):
        mx = lax.axis_index("x")
        my = lax.axis_index("y")
        mz = lax.axis_index("z")
        xo = 1 - mx

        x_dev = (xo, my, mz)

        def y_dev(j):
            return (mx, j, mz)

        def z_dev(j):
            return (mx, my, j)

        y_ring = (my, y_dev)
        z_ring = (mz, z_dev)

        barrier = pltpu.get_barrier_semaphore()
        for d in range(1, S4):
            pl.semaphore_signal(
                barrier, inc=1, device_id=y_dev((my + d) % S4),
                device_id_type=pl.DeviceIdType.MESH,
            )
            pl.semaphore_signal(
                barrier, inc=1, device_id=z_dev((mz + d) % S4),
                device_id_type=pl.DeviceIdType.MESH,
            )
        pl.semaphore_wait(barrier, 2 * (S4 - 1))

        f32 = jnp.float32
        bf16 = jnp.bfloat16

        def make_half(base, ring1, ring2, send_buf, recv_buf, xrecv,
                      ssems, rsems, cred):
            p1, dev1 = ring1
            p2, dev2 = ring2
            off1 = base + p1 * C1
            off2 = off1 + p2 * C2
            off3 = off2 + mx * C3

            def mates(p, dev):
                return [dev((p + d) % S4) for d in range(1, S4)]

            def sig(ci, devs):
                for dv in devs:
                    pl.semaphore_signal(
                        cred.at[ci], inc=1, device_id=dv,
                        device_id_type=pl.DeviceIdType.MESH,
                    )

            def rs4(P, p, dev, boff, c, src_is_x, nxt_devs):
                src = x_ref if src_is_x else out_ref

                def stage(d):
                    q = (p + d) % S4
                    send_buf[d - 1, pl.ds(0, c), :] = src[
                        pl.ds(boff + q * c, c), :].astype(bf16)

                def launch():
                    stage(1)
                    if P > 0:
                        pl.semaphore_wait(cred.at[P], S4 - 1)
                    rdmas = []
                    for d in range(1, S4):
                        if d > 1:
                            stage(d)
                        r = pltpu.make_async_remote_copy(
                            src_ref=send_buf.at[d - 1, pl.ds(0, c), :],
                            dst_ref=recv_buf.at[d - 1, pl.ds(0, c), :],
                            send_sem=ssems.at[d - 1],
                            recv_sem=rsems.at[d - 1],
                            device_id=dev((p + d) % S4),
                            device_id_type=pl.DeviceIdType.MESH,
                        )
                        r.start()
                        rdmas.append(r)
                    return rdmas

                def consume():
                    own = boff + p * c
                    acc = src[pl.ds(own, c), :]
                    for d in range(1, S4):
                        acc = acc + recv_buf[d - 1, pl.ds(0, c), :].astype(f32)
                    out_ref[pl.ds(own, c), :] = acc

                return launch, consume, (lambda: sig(P + 1, nxt_devs))

            def ag4(P, p, dev, boff, c, nxt_devs):
                def launch():
                    send_buf[0, pl.ds(0, c), :] = out_ref[
                        pl.ds(boff + p * c, c), :].astype(bf16)
                    pl.semaphore_wait(cred.at[P], S4 - 1)
                    rdmas = []
                    for d in range(1, S4):
                        r = pltpu.make_async_remote_copy(
                            src_ref=send_buf.at[0, pl.ds(0, c), :],
                            dst_ref=recv_buf.at[d - 1, pl.ds(0, c), :],
                            send_sem=ssems.at[d - 1],
                            recv_sem=rsems.at[d - 1],
                            device_id=dev((p + d) % S4),
                            device_id_type=pl.DeviceIdType.MESH,
                        )
                        r.start()
                        rdmas.append(r)
                    return rdmas

                def consume():
                    for d in range(1, S4):
                        s = (p - d) % S4
                        out_ref[pl.ds(boff + s * c, c), :] = recv_buf[
                            d - 1, pl.ds(0, c), :].astype(f32)

                def credit_sig():
                    if nxt_devs is not None:
                        sig(P + 1, nxt_devs)

                return launch, consume, credit_sig

            def x_rs(P):
                def launch():
                    pl.semaphore_wait(cred.at[P], 1)
                    r = pltpu.make_async_remote_copy(
                        src_ref=out_ref.at[pl.ds(off2 + xo * C3, C3), :],
                        dst_ref=xrecv,
                        send_sem=ssems.at[0],
                        recv_sem=rsems.at[0],
                        device_id=x_dev,
                        device_id_type=pl.DeviceIdType.MESH,
                    )
                    r.start()
                    return [r]

                def consume():
                    own = off2 + mx * C3
                    out_ref[pl.ds(own, C3), :] = (
                        out_ref[pl.ds(own, C3), :] + xrecv[...])

                return launch, consume, (lambda: sig(P + 1, [x_dev]))

            def x_ag(P, nxt_devs):
                def launch():
                    pl.semaphore_wait(cred.at[P], 1)
                    r = pltpu.make_async_remote_copy(
                        src_ref=out_ref.at[pl.ds(off3, C3), :],
                        dst_ref=out_ref.at[pl.ds(off3, C3), :],
                        send_sem=ssems.at[0],
                        recv_sem=rsems.at[0],
                        device_id=x_dev,
                        device_id_type=pl.DeviceIdType.MESH,
                    )
                    r.start()
                    return [r]

                return launch, (lambda: None), (lambda: sig(P + 1, nxt_devs))

            m1 = mates(p1, dev1)
            m2 = mates(p2, dev2)
            return [
                rs4(0, p1, dev1, base, C1, True, m2),
                rs4(1, p2, dev2, off1, C2, False, [x_dev]),
                x_rs(2),
                x_ag(3, m2),
                ag4(4, p2, dev2, off1, C2, m1),
                ag4(5, p1, dev1, base, C1, None),
            ]

        half_a = make_half(0, y_ring, z_ring, send_a, recv_a, xrecv_a,
                           ssem_a, rsem_a, cred_a)
        half_b = make_half(HALF, z_ring, y_ring, send_b, recv_b, xrecv_b,
                           ssem_b, rsem_b, cred_b)

        for pa, pb in zip(half_a, half_b):
            go_a, con_a, cs_a = pa
            go_b, con_b, cs_b = pb
            ras = go_a()
            rbs = go_b()
            for r in ras:
                r.wait()
            for r in rbs:
                r.wait()
            con_a()
            con_b()
            cs_a()
            cs_b()

    return pl.pallas_call(
        body,
        out_shape=jax.ShapeDtypeStruct((M, N), jnp.float32),
        in_specs=[pl.BlockSpec(memory_space=pltpu.VMEM)],
        out_specs=pl.BlockSpec(memory_space=pltpu.VMEM),
        scratch_shapes=[
            pltpu.VMEM((3, C1, N), jnp.bfloat16),
            pltpu.VMEM((3, C1, N), jnp.bfloat16),
            pltpu.VMEM((3, C1, N), jnp.bfloat16),
            pltpu.VMEM((3, C1, N), jnp.bfloat16),
            pltpu.VMEM((C3, N), jnp.float32),
            pltpu.VMEM((C3, N), jnp.float32),
            pltpu.SemaphoreType.DMA((3,)),
            pltpu.SemaphoreType.DMA((3,)),
            pltpu.SemaphoreType.DMA((3,)),
            pltpu.SemaphoreType.DMA((3,)),
            pltpu.SemaphoreType.REGULAR((N_PHASES,)),
            pltpu.SemaphoreType.REGULAR((N_PHASES,)),
        ],
        compiler_params=pltpu.CompilerParams(collective_id=0),
    )(x2)


# device time: 89256 ns/iter; 1.0036x vs baseline; 1.0036x over previous
import jax
import jax.numpy as jnp
from jax import lax
from jax.experimental import pallas as pl
from jax.experimental.pallas import tpu as pltpu

M, N = 2048, 1024
HALF = M // 2
S4 = 4
C1 = HALF // S4
C2 = C1 // S4
C3 = C2 // 2
N_PHASES = 6


def kernel(x):
    x2 = x.reshape(M, N)

    def body(x_ref, out_ref,
             send_a, recv_a, send_b, recv_b, xrecv_a, xrecv_b,
             ssem_a, rsem_a, ssem_b, rsem_b,
             cred_a, cred_b):
        mx = lax.axis_index("x")
        my = lax.axis_index("y")
        mz = lax.axis_index("z")
        xo = 1 - mx

        x_dev = (xo, my, mz)

        def y_dev(j):
            return (mx, j, mz)

        def z_dev(j):
            return (mx, my, j)

        y_ring = (my, y_dev)
        z_ring = (mz, z_dev)

        barrier = pltpu.get_barrier_semaphore()
        for d in range(1, S4):
            pl.semaphore_signal(
                barrier, inc=1, device_id=y_dev((my + d) % S4),
                device_id_type=pl.DeviceIdType.MESH,
            )
            pl.semaphore_signal(
                barrier, inc=1, device_id=z_dev((mz + d) % S4),
                device_id_type=pl.DeviceIdType.MESH,
            )
        pl.semaphore_wait(barrier, 2 * (S4 - 1))

        f32 = jnp.float32
        bf16 = jnp.bfloat16

        def make_half(base, ring1, ring2, send_buf, recv_buf, xrecv,
                      ssems, rsems, cred):
            p1, dev1 = ring1
            p2, dev2 = ring2
            off1 = base + p1 * C1
            off2 = off1 + p2 * C2
            off3 = off2 + mx * C3

            def mates(p, dev):
                return [dev((p + d) % S4) for d in range(1, S4)]

            def sig(ci, devs):
                for dv in devs:
                    pl.semaphore_signal(
                        cred.at[ci], inc=1, device_id=dv,
                        device_id_type=pl.DeviceIdType.MESH,
                    )

            def rs4(P, p, dev, boff, c, src_is_x, nxt_devs):
                src = x_ref if src_is_x else out_ref

                def stage(d):
                    q = (p + d) % S4
                    send_buf[d - 1, pl.ds(0, c), :] = src[
                        pl.ds(boff + q * c, c), :].astype(bf16)

                def launch():
                    stage(1)
                    if P > 0:
                        pl.semaphore_wait(cred.at[P], S4 - 1)
                    rdmas = []
                    for d in range(1, S4):
                        if d > 1:
                            stage(d)
                        r = pltpu.make_async_remote_copy(
                            src_ref=send_buf.at[d - 1, pl.ds(0, c), :],
                            dst_ref=recv_buf.at[d - 1, pl.ds(0, c), :],
                            send_sem=ssems.at[d - 1],
                            recv_sem=rsems.at[d - 1],
                            device_id=dev((p + d) % S4),
                            device_id_type=pl.DeviceIdType.MESH,
                        )
                        r.start()
                        rdmas.append(r)
                    return rdmas

                def consume_slot(d):
                    own = boff + p * c
                    inc = recv_buf[d - 1, pl.ds(0, c), :].astype(f32)
                    prev = src if d == 1 else out_ref
                    out_ref[pl.ds(own, c), :] = prev[pl.ds(own, c), :] + inc

                return launch, consume_slot, (lambda: sig(P + 1, nxt_devs))

            def ag4(P, p, dev, boff, c, nxt_devs):
                def launch():
                    send_buf[0, pl.ds(0, c), :] = out_ref[
                        pl.ds(boff + p * c, c), :].astype(bf16)
                    pl.semaphore_wait(cred.at[P], S4 - 1)
                    rdmas = []
                    for d in range(1, S4):
                        r = pltpu.make_async_remote_copy(
                            src_ref=send_buf.at[0, pl.ds(0, c), :],
                            dst_ref=recv_buf.at[d - 1, pl.ds(0, c), :],
                            send_sem=ssems.at[d - 1],
                            recv_sem=rsems.at[d - 1],
                            device_id=dev((p + d) % S4),
                            device_id_type=pl.DeviceIdType.MESH,
                        )
                        r.start()
                        rdmas.append(r)
                    return rdmas

                def consume_slot(d):
                    s = (p - d) % S4
                    out_ref[pl.ds(boff + s * c, c), :] = recv_buf[
                        d - 1, pl.ds(0, c), :].astype(f32)

                def credit_sig():
                    if nxt_devs is not None:
                        sig(P + 1, nxt_devs)

                return launch, consume_slot, credit_sig

            def x_rs(P):
                def launch():
                    pl.semaphore_wait(cred.at[P], 1)
                    r = pltpu.make_async_remote_copy(
                        src_ref=out_ref.at[pl.ds(off2 + xo * C3, C3), :],
                        dst_ref=xrecv,
                        send_sem=ssems.at[0],
                        recv_sem=rsems.at[0],
                        device_id=x_dev,
                        device_id_type=pl.DeviceIdType.MESH,
                    )
                    r.start()
                    return [r]

                def consume_slot(d):
                    own = off2 + mx * C3
                    out_ref[pl.ds(own, C3), :] = (
                        out_ref[pl.ds(own, C3), :] + xrecv[...])

                return launch, consume_slot, (lambda: sig(P + 1, [x_dev]))

            def x_ag(P, nxt_devs):
                def launch():
                    pl.semaphore_wait(cred.at[P], 1)
                    r = pltpu.make_async_remote_copy(
                        src_ref=out_ref.at[pl.ds(off3, C3), :],
                        dst_ref=out_ref.at[pl.ds(off3, C3), :],
                        send_sem=ssems.at[0],
                        recv_sem=rsems.at[0],
                        device_id=x_dev,
                        device_id_type=pl.DeviceIdType.MESH,
                    )
                    r.start()
                    return [r]

                return launch, (lambda d: None), (lambda: sig(P + 1, nxt_devs))

            m1 = mates(p1, dev1)
            m2 = mates(p2, dev2)
            return [
                rs4(0, p1, dev1, base, C1, True, m2),
                rs4(1, p2, dev2, off1, C2, False, [x_dev]),
                x_rs(2),
                x_ag(3, m2),
                ag4(4, p2, dev2, off1, C2, m1),
                ag4(5, p1, dev1, base, C1, None),
            ]

        half_a = make_half(0, y_ring, z_ring, send_a, recv_a, xrecv_a,
                           ssem_a, rsem_a, cred_a)
        half_b = make_half(HALF, z_ring, y_ring, send_b, recv_b, xrecv_b,
                           ssem_b, rsem_b, cred_b)

        for pa, pb in zip(half_a, half_b):
            go_a, con_a, cs_a = pa
            go_b, con_b, cs_b = pb
            ras = go_a()
            rbs = go_b()
            for d, r in enumerate(ras, start=1):
                r.wait()
                con_a(d)
            for d, r in enumerate(rbs, start=1):
                r.wait()
                con_b(d)
            cs_a()
            cs_b()

    return pl.pallas_call(
        body,
        out_shape=jax.ShapeDtypeStruct((M, N), jnp.float32),
        in_specs=[pl.BlockSpec(memory_space=pltpu.VMEM)],
        out_specs=pl.BlockSpec(memory_space=pltpu.VMEM),
        scratch_shapes=[
            pltpu.VMEM((3, C1, N), jnp.bfloat16),
            pltpu.VMEM((3, C1, N), jnp.bfloat16),
            pltpu.VMEM((3, C1, N), jnp.bfloat16),
            pltpu.VMEM((3, C1, N), jnp.bfloat16),
            pltpu.VMEM((C3, N), jnp.float32),
            pltpu.VMEM((C3, N), jnp.float32),
            pltpu.SemaphoreType.DMA((3,)),
            pltpu.SemaphoreType.DMA((3,)),
            pltpu.SemaphoreType.DMA((3,)),
            pltpu.SemaphoreType.DMA((3,)),
            pltpu.SemaphoreType.REGULAR((N_PHASES,)),
            pltpu.SemaphoreType.REGULAR((N_PHASES,)),
        ],
        compiler_params=pltpu.CompilerParams(collective_id=0),
    )(x2)


# device time: 86902 ns/iter; 1.0307x vs baseline; 1.0271x over previous
import jax
import jax.numpy as jnp
from jax import lax
from jax.experimental import pallas as pl
from jax.experimental.pallas import tpu as pltpu

M, N = 2048, 1024
HALF = M // 2
S4 = 4
C1 = HALF // S4
C2 = C1 // S4
C3 = C2 // 2
N_PHASES = 6


def kernel(x):
    x2 = x.reshape(M, N)

    def body(x_ref, out_ref,
             send_a, recv_a, send_b, recv_b,
             ssem_a, rsem_a, ssem_b, rsem_b,
             cred_a, cred_b):
        mx = lax.axis_index("x")
        my = lax.axis_index("y")
        mz = lax.axis_index("z")
        xo = 1 - mx

        x_dev = (xo, my, mz)

        def y_dev(j):
            return (mx, j, mz)

        def z_dev(j):
            return (mx, my, j)

        y_ring = (my, y_dev)
        z_ring = (mz, z_dev)

        barrier = pltpu.get_barrier_semaphore()
        for d in range(1, S4):
            pl.semaphore_signal(
                barrier, inc=1, device_id=y_dev((my + d) % S4),
                device_id_type=pl.DeviceIdType.MESH,
            )
            pl.semaphore_signal(
                barrier, inc=1, device_id=z_dev((mz + d) % S4),
                device_id_type=pl.DeviceIdType.MESH,
            )
        pl.semaphore_wait(barrier, 2 * (S4 - 1))

        f32 = jnp.float32
        bf16 = jnp.bfloat16

        def make_half(base, ring1, ring2, send_buf, recv_buf, ssems, rsems, cred):
            p1, dev1 = ring1
            p2, dev2 = ring2
            off1 = base + p1 * C1
            off2 = off1 + p2 * C2
            off3 = off2 + mx * C3

            def mates(p, dev):
                return [dev((p + d) % S4) for d in range(1, S4)]

            def sig(ci, devs):
                for dv in devs:
                    pl.semaphore_signal(
                        cred.at[ci], inc=1, device_id=dv,
                        device_id_type=pl.DeviceIdType.MESH,
                    )

            def starts(rows, targets):
                rdmas = []
                for slot, dv in targets:
                    r = pltpu.make_async_remote_copy(
                        src_ref=send_buf.at[slot, pl.ds(0, rows), :],
                        dst_ref=recv_buf.at[slot, pl.ds(0, rows), :],
                        send_sem=ssems.at[slot],
                        recv_sem=rsems.at[slot],
                        device_id=dv,
                        device_id_type=pl.DeviceIdType.MESH,
                    )
                    r.start()
                    rdmas.append(r)
                return rdmas

            def rs4(P, p, dev, boff, c, src_is_x, nxt_devs):
                src = x_ref if src_is_x else out_ref

                def stage():
                    for d in range(1, S4):
                        q = (p + d) % S4
                        send_buf[d - 1, pl.ds(0, c), :] = src[
                            pl.ds(boff + q * c, c), :].astype(bf16)

                def credit_wait():
                    if P > 0:
                        pl.semaphore_wait(cred.at[P], S4 - 1)

                def start():
                    return starts(c, [(d - 1, dev((p + d) % S4))
                                      for d in range(1, S4)])

                def consume():
                    own = boff + p * c
                    acc = src[pl.ds(own, c), :]
                    for d in range(1, S4):
                        acc = acc + recv_buf[d - 1, pl.ds(0, c), :].astype(f32)
                    out_ref[pl.ds(own, c), :] = acc

                def credit_sig():
                    sig(P + 1, nxt_devs)

                return stage, credit_wait, start, consume, credit_sig

            def ag4(P, p, dev, boff, c, nxt_devs):
                def stage():
                    send_buf[0, pl.ds(0, c), :] = out_ref[
                        pl.ds(boff + p * c, c), :].astype(bf16)

                def credit_wait():
                    pl.semaphore_wait(cred.at[P], S4 - 1)

                def start():
                    rdmas = []
                    for d in range(1, S4):
                        r = pltpu.make_async_remote_copy(
                            src_ref=send_buf.at[0, pl.ds(0, c), :],
                            dst_ref=recv_buf.at[d - 1, pl.ds(0, c), :],
                            send_sem=ssems.at[d - 1],
                            recv_sem=rsems.at[d - 1],
                            device_id=dev((p + d) % S4),
                            device_id_type=pl.DeviceIdType.MESH,
                        )
                        r.start()
                        rdmas.append(r)
                    return rdmas

                def consume():
                    for d in range(1, S4):
                        s = (p - d) % S4
                        out_ref[pl.ds(boff + s * c, c), :] = recv_buf[
                            d - 1, pl.ds(0, c), :].astype(f32)

                def credit_sig():
                    if nxt_devs is not None:
                        sig(P + 1, nxt_devs)

                return stage, credit_wait, start, consume, credit_sig

            def x_rs(P, nxt_devs):
                def stage():
                    send_buf[0, pl.ds(0, C3), :] = out_ref[
                        pl.ds(off2 + xo * C3, C3), :].astype(bf16)

                def credit_wait():
                    pl.semaphore_wait(cred.at[P], 1)

                def start():
                    return starts(C3, [(0, x_dev)])

                def consume():
                    own = off2 + mx * C3
                    out_ref[pl.ds(own, C3), :] = (
                        out_ref[pl.ds(own, C3), :]
                        + recv_buf[0, pl.ds(0, C3), :].astype(f32))

                def credit_sig():
                    sig(P + 1, [x_dev])

                return stage, credit_wait, start, consume, credit_sig

            def x_ag(P, nxt_devs):
                def stage():
                    send_buf[0, pl.ds(0, C3), :] = out_ref[
                        pl.ds(off3, C3), :].astype(bf16)

                def credit_wait():
                    pl.semaphore_wait(cred.at[P], 1)

                def start():
                    return starts(C3, [(0, x_dev)])

                def consume():
                    out_ref[pl.ds(off2 + xo * C3, C3), :] = recv_buf[
                        0, pl.ds(0, C3), :].astype(f32)

                def credit_sig():
                    sig(P + 1, nxt_devs)

                return stage, credit_wait, start, consume, credit_sig

            m1 = mates(p1, dev1)
            m2 = mates(p2, dev2)
            return [
                rs4(0, p1, dev1, base, C1, True, m2),
                rs4(1, p2, dev2, off1, C2, False, [x_dev]),
                x_rs(2, [x_dev]),
                x_ag(3, m2),
                ag4(4, p2, dev2, off1, C2, m1),
                ag4(5, p1, dev1, base, C1, None),
            ]

        half_a = make_half(0, y_ring, z_ring, send_a, recv_a,
                           ssem_a, rsem_a, cred_a)
        half_b = make_half(HALF, z_ring, y_ring, send_b, recv_b,
                           ssem_b, rsem_b, cred_b)

        for pa, pb in zip(half_a, half_b):
            st_a, cw_a, go_a, con_a, cs_a = pa
            st_b, cw_b, go_b, con_b, cs_b = pb
            st_a()
            st_b()
            cw_a()
            ras = go_a()
            cw_b()
            rbs = go_b()
            for r in ras:
                r.wait()
            for r in rbs:
                r.wait()
            con_a()
            con_b()
            cs_a()
            cs_b()

    return pl.pallas_call(
        body,
        out_shape=jax.ShapeDtypeStruct((M, N), jnp.float32),
        in_specs=[pl.BlockSpec(memory_space=pltpu.VMEM)],
        out_specs=pl.BlockSpec(memory_space=pltpu.VMEM),
        scratch_shapes=[
            pltpu.VMEM((3, C1, N), jnp.bfloat16),
            pltpu.VMEM((3, C1, N), jnp.bfloat16),
            pltpu.VMEM((3, C1, N), jnp.bfloat16),
            pltpu.VMEM((3, C1, N), jnp.bfloat16),
            pltpu.SemaphoreType.DMA((3,)),
            pltpu.SemaphoreType.DMA((3,)),
            pltpu.SemaphoreType.DMA((3,)),
            pltpu.SemaphoreType.DMA((3,)),
            pltpu.SemaphoreType.REGULAR((N_PHASES,)),
            pltpu.SemaphoreType.REGULAR((N_PHASES,)),
        ],
        compiler_params=pltpu.CompilerParams(collective_id=0),
    )(x2)
